# baseline (speedup 1.0000x reference)
import jax
import jax.numpy as jnp
from jax import lax
from jax.experimental import pallas as pl
from jax.experimental.pallas import tpu as pltpu
from jax.experimental.pallas import tpu_sc as plsc

N_USERS = 50000
N_NODES = 100000
HD = 16
N_LAYERS = 3
N_EDGES = 1600000
BATCH = 4096
B2 = 2 * BATCH

NC = 2
NS = 16
L = 16

EPT = N_EDGES // NS
EC = 800
N_ECHUNK = EPT // EC

RC = 800
N_RCHUNK = N_NODES // RC
BPT = B2 // NS
BC = 128


def _vrsqrt(x):
    i = lax.bitcast_convert_type(x, jnp.int32)
    i = jnp.int32(0x5F3759DF) - lax.shift_right_logical(i, 1)
    y = lax.bitcast_convert_type(i, jnp.float32)
    for _ in range(3):
        y = y * (jnp.float32(1.5) - jnp.float32(0.5) * x * y * y)
    return y


def _n_chunks(s):
    return jnp.where(s < N_RCHUNK % NS, N_RCHUNK // NS + 1, N_RCHUNK // NS)


def _scale_inplace(buf, s16_fn, nrows):
    def grp(g, _):
        s16 = s16_fn(g)
        for r in range(L):
            buf[g * L + r, :] = buf[g * L + r, :] * s16[r]
        return 0
    lax.fori_loop(0, nrows // L, grp, 0)


def _zero_fill(buf, nrows):
    def grp(g, _):
        for r in range(L):
            buf[g * L + r, :] = jnp.zeros((L,), jnp.float32)
        return 0
    lax.fori_loop(0, nrows // L, grp, 0)


GSPLIT = 4


def _gather_issue(xt_hbm, idx, msg, sems):
    w = EC // GSPLIT
    for k in range(GSPLIT):
        pltpu.async_copy(xt_hbm.at[idx.at[pl.ds(k * w, w)]],
                         msg.at[pl.ds(k * w, w), :], sems[k])


def _gather_wait(xt_hbm, idx, msg, sems):
    w = EC // GSPLIT
    for k in range(GSPLIT):
        pltpu.make_async_copy(xt_hbm.at[idx.at[pl.ds(k * w, w)]],
                              msg.at[pl.ds(k * w, w), :], sems[k]).wait()


def _deg_body(edges_ref, ab_ref, deg_sh, idx_v, idx_w, ones_v, vbuf, obuf,
              si0, si1):
    c = lax.axis_index("c")
    s = lax.axis_index("s")

    def init_ones(i, _):
        ones_v[pl.ds(i * L, L)] = jnp.ones((L,), jnp.float32)
        return 0
    lax.fori_loop(0, EC // L, init_ones, 0)

    def init_zero(i, _):
        obuf[pl.ds(i * L, L)] = jnp.zeros((L,), jnp.float32)
        return 0
    lax.fori_loop(0, RC // L, init_zero, 0)

    def zero_chunk(j, _):
        cid = s + NS * j
        pltpu.sync_copy(obuf, deg_sh.at[pl.ds(cid * RC, RC)])
        return 0
    lax.fori_loop(0, _n_chunks(s), zero_chunk, 0)

    plsc.subcore_barrier()

    ebase = c * N_EDGES + s * EPT
    pltpu.async_copy(edges_ref.at[pl.ds(ebase, EC)], idx_v, si0)
    pltpu.async_copy(edges_ref.at[pl.ds(ebase + EC, EC)], idx_w, si1)

    def deg_pair(j, _):
        b0 = ebase + 2 * j * EC
        pltpu.make_async_copy(edges_ref.at[pl.ds(b0, EC)], idx_v,
                              si0).wait()
        pltpu.sync_copy(ones_v, deg_sh.at[idx_v], add=True)
        pltpu.async_copy(edges_ref.at[pl.ds(b0 + 2 * EC, EC)], idx_v, si0)
        pltpu.make_async_copy(edges_ref.at[pl.ds(b0 + EC, EC)], idx_w,
                              si1).wait()
        pltpu.sync_copy(ones_v, deg_sh.at[idx_w], add=True)
        @pl.when(j < N_ECHUNK // 2 - 1)
        def _():
            pltpu.async_copy(edges_ref.at[pl.ds(b0 + 3 * EC, EC)], idx_w,
                             si1)
        return 0
    lax.fori_loop(0, N_ECHUNK // 2, deg_pair, 0)
    pltpu.make_async_copy(
        edges_ref.at[pl.ds(ebase + (N_ECHUNK - 1) * EC, EC)], idx_v,
        si0).wait()
    pltpu.sync_copy(ones_v, deg_sh.at[idx_v], add=True)

    plsc.subcore_barrier()

    def conv_chunk(j, _):
        cid = s + NS * j
        pltpu.sync_copy(deg_sh.at[pl.ds(cid * RC, RC)], vbuf)
        def conv_grp(g, _):
            x = jnp.maximum(vbuf[pl.ds(g * L, L)], jnp.float32(1.0))
            obuf[pl.ds(g * L, L)] = _vrsqrt(x)
            return 0
        lax.fori_loop(0, RC // L, conv_grp, 0)
        pltpu.sync_copy(obuf, ab_ref.at[pl.ds(c * N_NODES + cid * RC, RC)])
        return 0
    lax.fori_loop(0, _n_chunks(s), conv_chunk, 0)


def _main_body(x0_ref, srcoff_ref, dst_ref, ab_ref, b_ref, bidx2_ref,
               out_ref, acc_sh, xt_hbm,
               idx_s0, idx_s1, idx_d0, idx_d1, msg0, msg1, abuf, bbuf,
               g0a, g0b, g0c, g0d, g1a, g1b, g1c, g1d, si0, si1):
    c = lax.axis_index("c")
    s = lax.axis_index("s")
    sgA = (g0a, g0b, g0c, g0d)
    sgB = (g1a, g1b, g1c, g1d)
    dchunk = msg0
    btmp = msg1.at[pl.ds(0, BC), :]
    bchunk = msg1.at[pl.ds(BC, BC), :]
    bidx_v = idx_s0.at[pl.ds(0, BC)]
    bvals = bbuf.at[pl.ds(0, BC)]

    def p1_chunk(j, _):
        cid = s + NS * j
        n0 = cid * RC
        r0 = c * N_NODES + n0
        pltpu.sync_copy(ab_ref.at[pl.ds(n0, RC)], abuf)
        pltpu.sync_copy(x0_ref.at[pl.ds(r0, RC), :], dchunk)
        _scale_inplace(dchunk, lambda g: abuf[pl.ds(g * L, L)], RC)
        pltpu.sync_copy(dchunk, xt_hbm.at[pl.ds(r0, RC), :])
        _zero_fill(dchunk, RC)
        pltpu.sync_copy(dchunk, acc_sh.at[pl.ds(n0, RC), :])
        return 0
    lax.fori_loop(0, _n_chunks(s), p1_chunk, 0)

    for m in range(BPT // BC):
        bb = s * BPT + m * BC
        pltpu.sync_copy(bidx2_ref.at[pl.ds(c * B2 + bb, BC)], bidx_v)
        pltpu.sync_copy(x0_ref.at[bidx_v], bchunk)
        pltpu.sync_copy(bchunk, out_ref.at[pl.ds(c * B2 + bb, BC), :])

    plsc.subcore_barrier()

    sbase = s * EPT
    coff = c * N_EDGES
    for layer in range(N_LAYERS):
        pltpu.sync_copy(srcoff_ref.at[pl.ds(coff + sbase, EC)], idx_s0)
        pltpu.sync_copy(dst_ref.at[pl.ds(sbase, EC)], idx_d0)
        _gather_issue(xt_hbm, idx_s0, msg0, sgA)
        pltpu.async_copy(srcoff_ref.at[pl.ds(coff + sbase + EC, EC)],
                         idx_s1, si1)
        pltpu.async_copy(dst_ref.at[pl.ds(sbase + EC, EC)], idx_d1, si1)

        def edge_body(j, _):
            base0 = sbase + 2 * j * EC
            not_last = j < N_ECHUNK // 2 - 1
            pltpu.make_async_copy(
                srcoff_ref.at[pl.ds(coff + base0 + EC, EC)], idx_s1,
                si1).wait()
            pltpu.make_async_copy(
                dst_ref.at[pl.ds(base0 + EC, EC)], idx_d1, si1).wait()
            _gather_wait(xt_hbm, idx_s0, msg0, sgA)
            _gather_issue(xt_hbm, idx_s1, msg1, sgB)
            pltpu.sync_copy(msg0, acc_sh.at[idx_d0], add=True)
            pltpu.async_copy(
                srcoff_ref.at[pl.ds(coff + base0 + 2 * EC, EC)],
                idx_s0, si0)
            pltpu.async_copy(dst_ref.at[pl.ds(base0 + 2 * EC, EC)],
                             idx_d0, si0)
            _gather_wait(xt_hbm, idx_s1, msg1, sgB)
            pltpu.make_async_copy(
                srcoff_ref.at[pl.ds(coff + base0 + 2 * EC, EC)],
                idx_s0, si0).wait()
            pltpu.make_async_copy(
                dst_ref.at[pl.ds(base0 + 2 * EC, EC)], idx_d0, si0).wait()
            _gather_issue(xt_hbm, idx_s0, msg0, sgA)
            pltpu.sync_copy(msg1, acc_sh.at[idx_d1], add=True)
            @pl.when(not_last)
            def _():
                pltpu.async_copy(
                    srcoff_ref.at[pl.ds(coff + base0 + 3 * EC, EC)],
                    idx_s1, si1)
                pltpu.async_copy(dst_ref.at[pl.ds(base0 + 3 * EC, EC)],
                                 idx_d1, si1)
            return 0
        lax.fori_loop(0, N_ECHUNK // 2, edge_body, 0)
        _gather_wait(xt_hbm, idx_s0, msg0, sgA)
        pltpu.sync_copy(msg0, acc_sh.at[idx_d0], add=True)

        plsc.subcore_barrier()

        for m in range(BPT // BC):
            bb = s * BPT + m * BC
            pltpu.sync_copy(bidx2_ref.at[pl.ds(bb, BC)], bidx_v)
            pltpu.sync_copy(acc_sh.at[bidx_v], btmp)
            pltpu.sync_copy(b_ref.at[bidx_v], bvals)
            pltpu.sync_copy(out_ref.at[pl.ds(c * B2 + bb, BC), :], bchunk)
            def bacc_grp(g, _):
                b16 = bvals[pl.ds(g * L, L)]
                for r in range(L):
                    row = bchunk[g * L + r, :] + btmp[g * L + r, :] * b16[r]
                    if layer == N_LAYERS - 1:
                        row = row * jnp.float32(0.25)
                    bchunk[g * L + r, :] = row
                return 0
            lax.fori_loop(0, BC // L, bacc_grp, 0)
            pltpu.sync_copy(bchunk, out_ref.at[pl.ds(c * B2 + bb, BC), :])

        plsc.subcore_barrier()

        if layer < N_LAYERS - 1:
            def dense_chunk(j, _):
                cid = s + NS * j
                n0 = cid * RC
                r0 = c * N_NODES + n0
                pltpu.sync_copy(ab_ref.at[pl.ds(n0, RC)], abuf)
                pltpu.sync_copy(b_ref.at[pl.ds(n0, RC)], bbuf)
                pltpu.sync_copy(acc_sh.at[pl.ds(n0, RC), :], dchunk)
                _scale_inplace(
                    dchunk,
                    lambda g: abuf[pl.ds(g * L, L)] * bbuf[pl.ds(g * L, L)],
                    RC)
                pltpu.sync_copy(dchunk, xt_hbm.at[pl.ds(r0, RC), :])
                _zero_fill(dchunk, RC)
                pltpu.sync_copy(dchunk, acc_sh.at[pl.ds(n0, RC), :])
                return 0
            lax.fori_loop(0, _n_chunks(s), dense_chunk, 0)

            plsc.subcore_barrier()


@jax.jit
def _run(x0flat, edges_flat, srcoff, dst, bidx2):
    mesh = plsc.VectorSubcoreMesh(core_axis_name="c", subcore_axis_name="s")
    params = pltpu.CompilerParams(use_tc_tiling_on_sc=False)

    deg_k = pl.kernel(
        _deg_body,
        out_type=jax.ShapeDtypeStruct((2 * N_NODES,), jnp.float32),
        mesh=mesh,
        compiler_params=params,
        scratch_types=[
            pltpu.VMEM_SHARED((N_NODES,), jnp.float32),
            pltpu.VMEM((EC,), jnp.int32),
            pltpu.VMEM((EC,), jnp.int32),
            pltpu.VMEM((EC,), jnp.float32),
            pltpu.VMEM((RC,), jnp.float32),
            pltpu.VMEM((RC,), jnp.float32),
            pltpu.SemaphoreType.DMA,
            pltpu.SemaphoreType.DMA,
        ],
    )
    ab = deg_k(edges_flat)
    a_tab = ab[:N_NODES]
    b_tab = ab[N_NODES:]

    main_k = pl.kernel(
        _main_body,
        out_type=jax.ShapeDtypeStruct((NC * B2, HD), jnp.float32),
        mesh=mesh,
        compiler_params=params,
        scratch_types=[
            pltpu.VMEM_SHARED((N_NODES, HD), jnp.float32),
            pltpu.MemorySpace.HBM((NC * N_NODES, HD), jnp.float32),
            pltpu.VMEM((EC,), jnp.int32),
            pltpu.VMEM((EC,), jnp.int32),
            pltpu.VMEM((EC,), jnp.int32),
            pltpu.VMEM((EC,), jnp.int32),
            pltpu.VMEM((EC, HD), jnp.float32),
            pltpu.VMEM((EC, HD), jnp.float32),
            pltpu.VMEM((RC,), jnp.float32),
            pltpu.VMEM((RC,), jnp.float32),
            pltpu.SemaphoreType.DMA,
            pltpu.SemaphoreType.DMA,
            pltpu.SemaphoreType.DMA,
            pltpu.SemaphoreType.DMA,
            pltpu.SemaphoreType.DMA,
            pltpu.SemaphoreType.DMA,
            pltpu.SemaphoreType.DMA,
            pltpu.SemaphoreType.DMA,
            pltpu.SemaphoreType.DMA,
            pltpu.SemaphoreType.DMA,
        ],
    )
    return main_k(x0flat, srcoff, dst, a_tab, b_tab, bidx2)


def kernel(users, items, edge_index, user_emb, item_emb):
    src = edge_index[0].astype(jnp.int32)
    dst = edge_index[1].astype(jnp.int32)
    x0 = jnp.concatenate([user_emb, item_emb], axis=0)
    x0flat = jnp.concatenate([x0[:, :HD], x0[:, HD:]], axis=0)
    edges_flat = jnp.concatenate([src, dst])
    srcoff = jnp.concatenate([src, src + N_NODES])
    bidx = jnp.concatenate([users.astype(jnp.int32),
                            items.astype(jnp.int32) + N_USERS])
    bidx2 = jnp.concatenate([bidx, bidx + N_NODES])
    out = _run(x0flat, edges_flat, srcoff, dst, bidx2)
    users_emb = jnp.concatenate([out[:BATCH], out[B2:B2 + BATCH]], axis=1)
    items_emb = jnp.concatenate([out[BATCH:B2], out[B2 + BATCH:]], axis=1)
    return (users_emb, items_emb)


# speedup vs baseline: 1.0203x; 1.0203x over previous
import jax
import jax.numpy as jnp
from jax import lax
from jax.experimental import pallas as pl
from jax.experimental.pallas import tpu as pltpu
from jax.experimental.pallas import tpu_sc as plsc

N_USERS = 50000
N_NODES = 100000
HD = 16
N_LAYERS = 3
N_EDGES = 1600000
BATCH = 4096
B2 = 2 * BATCH

NC = 2
NS = 16
L = 16

EPT = N_EDGES // NS
EC = 800
N_ECHUNK = EPT // EC

RC = 800
N_RCHUNK = N_NODES // RC
BPT = B2 // NS
BC = 128


def _vrsqrt(x):
    i = lax.bitcast_convert_type(x, jnp.int32)
    i = jnp.int32(0x5F3759DF) - lax.shift_right_logical(i, 1)
    y = lax.bitcast_convert_type(i, jnp.float32)
    for _ in range(3):
        y = y * (jnp.float32(1.5) - jnp.float32(0.5) * x * y * y)
    return y


def _n_chunks(s):
    return jnp.where(s < N_RCHUNK % NS, N_RCHUNK // NS + 1, N_RCHUNK // NS)


def _scale_inplace(buf, s16_fn, nrows):
    def grp(g, _):
        s16 = s16_fn(g)
        for r in range(L):
            buf[g * L + r, :] = buf[g * L + r, :] * s16[r]
        return 0
    lax.fori_loop(0, nrows // L, grp, 0)


def _zero_fill(buf, nrows):
    def grp(g, _):
        for r in range(L):
            buf[g * L + r, :] = jnp.zeros((L,), jnp.float32)
        return 0
    lax.fori_loop(0, nrows // L, grp, 0)


def _deg_body(edges_ref, ab_ref, deg_sh, idx_v, idx_w, ones_v, vbuf, obuf,
              si0, si1, sd0, sd1):
    c = lax.axis_index("c")
    s = lax.axis_index("s")

    def init_ones(i, _):
        ones_v[pl.ds(i * L, L)] = jnp.ones((L,), jnp.float32)
        return 0
    lax.fori_loop(0, EC // L, init_ones, 0)

    def init_zero(i, _):
        obuf[pl.ds(i * L, L)] = jnp.zeros((L,), jnp.float32)
        return 0
    lax.fori_loop(0, RC // L, init_zero, 0)

    def zero_chunk(j, _):
        cid = s + NS * j
        pltpu.sync_copy(obuf, deg_sh.at[pl.ds(cid * RC, RC)])
        return 0
    lax.fori_loop(0, _n_chunks(s), zero_chunk, 0)

    plsc.subcore_barrier()

    ebase = c * N_EDGES + s * EPT
    pltpu.async_copy(edges_ref.at[pl.ds(ebase, EC)], idx_v, si0)
    pltpu.async_copy(edges_ref.at[pl.ds(ebase + EC, EC)], idx_w, si1)

    def _deg_scat(idx):
        h = EC // 2
        d1 = pltpu.async_copy(ones_v.at[pl.ds(0, h)],
                              deg_sh.at[idx.at[pl.ds(0, h)]], sd0,
                              add=True)
        d2 = pltpu.async_copy(ones_v.at[pl.ds(h, h)],
                              deg_sh.at[idx.at[pl.ds(h, h)]], sd1,
                              add=True)
        d1.wait()
        d2.wait()

    def deg_pair(j, _):
        b0 = ebase + 2 * j * EC
        pltpu.make_async_copy(edges_ref.at[pl.ds(b0, EC)], idx_v,
                              si0).wait()
        _deg_scat(idx_v)
        pltpu.async_copy(edges_ref.at[pl.ds(b0 + 2 * EC, EC)], idx_v, si0)
        pltpu.make_async_copy(edges_ref.at[pl.ds(b0 + EC, EC)], idx_w,
                              si1).wait()
        _deg_scat(idx_w)
        @pl.when(j < N_ECHUNK // 2 - 1)
        def _():
            pltpu.async_copy(edges_ref.at[pl.ds(b0 + 3 * EC, EC)], idx_w,
                             si1)
        return 0
    lax.fori_loop(0, N_ECHUNK // 2, deg_pair, 0)
    pltpu.make_async_copy(
        edges_ref.at[pl.ds(ebase + (N_ECHUNK - 1) * EC, EC)], idx_v,
        si0).wait()
    _deg_scat(idx_v)

    plsc.subcore_barrier()

    def conv_chunk(j, _):
        cid = s + NS * j
        pltpu.sync_copy(deg_sh.at[pl.ds(cid * RC, RC)], vbuf)
        def conv_grp(g, _):
            x = jnp.maximum(vbuf[pl.ds(g * L, L)], jnp.float32(1.0))
            obuf[pl.ds(g * L, L)] = _vrsqrt(x)
            return 0
        lax.fori_loop(0, RC // L, conv_grp, 0)
        pltpu.sync_copy(obuf, ab_ref.at[pl.ds(c * N_NODES + cid * RC, RC)])
        return 0
    lax.fori_loop(0, _n_chunks(s), conv_chunk, 0)


def _main_body(x0_ref, srcoff_ref, dst_ref, ab_ref, b_ref, bidx2_ref,
               out_ref, acc_sh, xt_hbm,
               idx_s0, idx_s1, idx_d0, idx_d1, msg0, msg1, abuf, bbuf,
               sg0, sg1, sg0b, sg1b, si0, si1):
    c = lax.axis_index("c")
    s = lax.axis_index("s")
    dchunk = msg0
    btmp = msg1.at[pl.ds(0, BC), :]
    bchunk = msg1.at[pl.ds(BC, BC), :]
    bidx_v = idx_s0.at[pl.ds(0, BC)]
    bidx_a = idx_s1.at[pl.ds(0, BPT)]
    bvals = bbuf.at[pl.ds(0, BC)]

    def p1_chunk(j, _):
        cid = s + NS * j
        n0 = cid * RC
        r0 = c * N_NODES + n0
        pltpu.sync_copy(ab_ref.at[pl.ds(n0, RC)], abuf)
        pltpu.sync_copy(x0_ref.at[pl.ds(r0, RC), :], dchunk)
        _scale_inplace(dchunk, lambda g: abuf[pl.ds(g * L, L)], RC)
        pltpu.sync_copy(dchunk, xt_hbm.at[pl.ds(r0, RC), :])
        _zero_fill(dchunk, RC)
        pltpu.sync_copy(dchunk, acc_sh.at[pl.ds(n0, RC), :])
        return 0
    lax.fori_loop(0, _n_chunks(s), p1_chunk, 0)

    for m in range(BPT // BC):
        bb = s * BPT + m * BC
        pltpu.sync_copy(bidx2_ref.at[pl.ds(c * B2 + bb, BC)], bidx_v)
        pltpu.sync_copy(x0_ref.at[bidx_v], bchunk)
        pltpu.sync_copy(bchunk, out_ref.at[pl.ds(c * B2 + bb, BC), :])

    plsc.subcore_barrier()

    sbase = s * EPT
    coff = c * N_EDGES
    for layer in range(N_LAYERS):
        pltpu.sync_copy(srcoff_ref.at[pl.ds(coff + sbase, EC)], idx_s0)
        pltpu.sync_copy(dst_ref.at[pl.ds(sbase, EC)], idx_d0)
        pltpu.async_copy(xt_hbm.at[idx_s0.at[pl.ds(0, EC // 2)]],
                         msg0.at[pl.ds(0, EC // 2), :], sg0)
        pltpu.async_copy(xt_hbm.at[idx_s0.at[pl.ds(EC // 2, EC // 2)]],
                         msg0.at[pl.ds(EC // 2, EC // 2), :], sg0b)
        pltpu.async_copy(srcoff_ref.at[pl.ds(coff + sbase + EC, EC)],
                         idx_s1, si1)
        pltpu.async_copy(dst_ref.at[pl.ds(sbase + EC, EC)], idx_d1, si1)

        def edge_body(j, _):
            base0 = sbase + 2 * j * EC
            not_last = j < N_ECHUNK // 2 - 1
            pltpu.make_async_copy(
                srcoff_ref.at[pl.ds(coff + base0 + EC, EC)], idx_s1,
                si1).wait()
            pltpu.make_async_copy(
                dst_ref.at[pl.ds(base0 + EC, EC)], idx_d1, si1).wait()
            pltpu.make_async_copy(
                xt_hbm.at[idx_s0.at[pl.ds(0, EC // 2)]],
                msg0.at[pl.ds(0, EC // 2), :], sg0).wait()
            pltpu.make_async_copy(
                xt_hbm.at[idx_s0.at[pl.ds(EC // 2, EC // 2)]],
                msg0.at[pl.ds(EC // 2, EC // 2), :], sg0b).wait()
            pltpu.async_copy(xt_hbm.at[idx_s1.at[pl.ds(0, EC // 2)]],
                             msg1.at[pl.ds(0, EC // 2), :], sg1)
            pltpu.async_copy(xt_hbm.at[idx_s1.at[pl.ds(EC // 2, EC // 2)]],
                             msg1.at[pl.ds(EC // 2, EC // 2), :], sg1b)
            pltpu.sync_copy(msg0, acc_sh.at[idx_d0], add=True)
            pltpu.async_copy(
                srcoff_ref.at[pl.ds(coff + base0 + 2 * EC, EC)],
                idx_s0, si0)
            pltpu.async_copy(dst_ref.at[pl.ds(base0 + 2 * EC, EC)],
                             idx_d0, si0)
            pltpu.make_async_copy(
                xt_hbm.at[idx_s1.at[pl.ds(0, EC // 2)]],
                msg1.at[pl.ds(0, EC // 2), :], sg1).wait()
            pltpu.make_async_copy(
                xt_hbm.at[idx_s1.at[pl.ds(EC // 2, EC // 2)]],
                msg1.at[pl.ds(EC // 2, EC // 2), :], sg1b).wait()
            pltpu.make_async_copy(
                srcoff_ref.at[pl.ds(coff + base0 + 2 * EC, EC)],
                idx_s0, si0).wait()
            pltpu.make_async_copy(
                dst_ref.at[pl.ds(base0 + 2 * EC, EC)], idx_d0, si0).wait()
            pltpu.async_copy(xt_hbm.at[idx_s0.at[pl.ds(0, EC // 2)]],
                             msg0.at[pl.ds(0, EC // 2), :], sg0)
            pltpu.async_copy(xt_hbm.at[idx_s0.at[pl.ds(EC // 2, EC // 2)]],
                             msg0.at[pl.ds(EC // 2, EC // 2), :], sg0b)
            pltpu.sync_copy(msg1, acc_sh.at[idx_d1], add=True)
            @pl.when(not_last)
            def _():
                pltpu.async_copy(
                    srcoff_ref.at[pl.ds(coff + base0 + 3 * EC, EC)],
                    idx_s1, si1)
                pltpu.async_copy(dst_ref.at[pl.ds(base0 + 3 * EC, EC)],
                                 idx_d1, si1)
            return 0
        lax.fori_loop(0, N_ECHUNK // 2, edge_body, 0)
        pltpu.make_async_copy(
            xt_hbm.at[idx_s0.at[pl.ds(0, EC // 2)]],
            msg0.at[pl.ds(0, EC // 2), :], sg0).wait()
        pltpu.make_async_copy(
            xt_hbm.at[idx_s0.at[pl.ds(EC // 2, EC // 2)]],
            msg0.at[pl.ds(EC // 2, EC // 2), :], sg0b).wait()
        pltpu.sync_copy(msg0, acc_sh.at[idx_d0], add=True)

        plsc.subcore_barrier()

        pltpu.sync_copy(bidx2_ref.at[pl.ds(s * BPT, BPT)], bidx_a)
        for m in range(BPT // BC):
            bb = s * BPT + m * BC
            bidx_m = bidx_a.at[pl.ds(m * BC, BC)]
            d1 = pltpu.async_copy(acc_sh.at[bidx_m], btmp, sg0)
            d2 = pltpu.async_copy(b_ref.at[bidx_m], bvals, sg1)
            d3 = pltpu.async_copy(out_ref.at[pl.ds(c * B2 + bb, BC), :],
                                  bchunk, si0)
            d1.wait()
            d2.wait()
            d3.wait()
            def bacc_grp(g, _):
                b16 = bvals[pl.ds(g * L, L)]
                for r in range(L):
                    row = bchunk[g * L + r, :] + btmp[g * L + r, :] * b16[r]
                    if layer == N_LAYERS - 1:
                        row = row * jnp.float32(0.25)
                    bchunk[g * L + r, :] = row
                return 0
            lax.fori_loop(0, BC // L, bacc_grp, 0)
            pltpu.sync_copy(bchunk, out_ref.at[pl.ds(c * B2 + bb, BC), :])

        plsc.subcore_barrier()

        if layer < N_LAYERS - 1:
            def dense_chunk(j, _):
                cid = s + NS * j
                n0 = cid * RC
                r0 = c * N_NODES + n0
                pltpu.sync_copy(ab_ref.at[pl.ds(n0, RC)], abuf)
                pltpu.sync_copy(b_ref.at[pl.ds(n0, RC)], bbuf)
                pltpu.sync_copy(acc_sh.at[pl.ds(n0, RC), :], dchunk)
                _scale_inplace(
                    dchunk,
                    lambda g: abuf[pl.ds(g * L, L)] * bbuf[pl.ds(g * L, L)],
                    RC)
                pltpu.sync_copy(dchunk, xt_hbm.at[pl.ds(r0, RC), :])
                _zero_fill(dchunk, RC)
                pltpu.sync_copy(dchunk, acc_sh.at[pl.ds(n0, RC), :])
                return 0
            lax.fori_loop(0, _n_chunks(s), dense_chunk, 0)

            plsc.subcore_barrier()


@jax.jit
def _run(x0flat, edges_flat, srcoff, dst, bidx2):
    mesh = plsc.VectorSubcoreMesh(core_axis_name="c", subcore_axis_name="s")
    params = pltpu.CompilerParams(use_tc_tiling_on_sc=False)

    deg_k = pl.kernel(
        _deg_body,
        out_type=jax.ShapeDtypeStruct((2 * N_NODES,), jnp.float32),
        mesh=mesh,
        compiler_params=params,
        scratch_types=[
            pltpu.VMEM_SHARED((N_NODES,), jnp.float32),
            pltpu.VMEM((EC,), jnp.int32),
            pltpu.VMEM((EC,), jnp.int32),
            pltpu.VMEM((EC,), jnp.float32),
            pltpu.VMEM((RC,), jnp.float32),
            pltpu.VMEM((RC,), jnp.float32),
            pltpu.SemaphoreType.DMA,
            pltpu.SemaphoreType.DMA,
            pltpu.SemaphoreType.DMA,
            pltpu.SemaphoreType.DMA,
        ],
    )
    ab = deg_k(edges_flat)
    a_tab = ab[:N_NODES]
    b_tab = ab[N_NODES:]

    main_k = pl.kernel(
        _main_body,
        out_type=jax.ShapeDtypeStruct((NC * B2, HD), jnp.float32),
        mesh=mesh,
        compiler_params=params,
        scratch_types=[
            pltpu.VMEM_SHARED((N_NODES, HD), jnp.float32),
            pltpu.MemorySpace.HBM((NC * N_NODES, HD), jnp.float32),
            pltpu.VMEM((EC,), jnp.int32),
            pltpu.VMEM((EC,), jnp.int32),
            pltpu.VMEM((EC,), jnp.int32),
            pltpu.VMEM((EC,), jnp.int32),
            pltpu.VMEM((EC, HD), jnp.float32),
            pltpu.VMEM((EC, HD), jnp.float32),
            pltpu.VMEM((RC,), jnp.float32),
            pltpu.VMEM((RC,), jnp.float32),
            pltpu.SemaphoreType.DMA,
            pltpu.SemaphoreType.DMA,
            pltpu.SemaphoreType.DMA,
            pltpu.SemaphoreType.DMA,
            pltpu.SemaphoreType.DMA,
            pltpu.SemaphoreType.DMA,
        ],
    )
    return main_k(x0flat, srcoff, dst, a_tab, b_tab, bidx2)


def kernel(users, items, edge_index, user_emb, item_emb):
    src = edge_index[0].astype(jnp.int32)
    dst = edge_index[1].astype(jnp.int32)
    x0 = jnp.concatenate([user_emb, item_emb], axis=0)
    x0flat = jnp.concatenate([x0[:, :HD], x0[:, HD:]], axis=0)
    edges_flat = jnp.concatenate([src, dst])
    srcoff = jnp.concatenate([src, src + N_NODES])
    bidx = jnp.concatenate([users.astype(jnp.int32),
                            items.astype(jnp.int32) + N_USERS])
    bidx2 = jnp.concatenate([bidx, bidx + N_NODES])
    out = _run(x0flat, edges_flat, srcoff, dst, bidx2)
    users_emb = jnp.concatenate([out[:BATCH], out[B2:B2 + BATCH]], axis=1)
    items_emb = jnp.concatenate([out[BATCH:B2], out[B2 + BATCH:]], axis=1)
    return (users_emb, items_emb)


# speedup vs baseline: 1.0401x; 1.0193x over previous
import jax
import jax.numpy as jnp
from jax import lax
from jax.experimental import pallas as pl
from jax.experimental.pallas import tpu as pltpu
from jax.experimental.pallas import tpu_sc as plsc

N_USERS = 50000
N_NODES = 100000
HD = 16
N_LAYERS = 3
N_EDGES = 1600000
BATCH = 4096
B2 = 2 * BATCH

NC = 2
NS = 16
L = 16

EPT = N_EDGES // NS
EC = 800
N_ECHUNK = EPT // EC

RC = 800
N_RCHUNK = N_NODES // RC
BPT = B2 // NS
BC = 128


def _vrsqrt(x):
    i = lax.bitcast_convert_type(x, jnp.int32)
    i = jnp.int32(0x5F3759DF) - lax.shift_right_logical(i, 1)
    y = lax.bitcast_convert_type(i, jnp.float32)
    for _ in range(3):
        y = y * (jnp.float32(1.5) - jnp.float32(0.5) * x * y * y)
    return y


def _n_chunks(s):
    return jnp.where(s < N_RCHUNK % NS, N_RCHUNK // NS + 1, N_RCHUNK // NS)


def _scale_inplace(buf, s16_fn, nrows):
    def grp(g, _):
        s16 = s16_fn(g)
        for r in range(L):
            buf[g * L + r, :] = buf[g * L + r, :] * s16[r]
        return 0
    lax.fori_loop(0, nrows // L, grp, 0)


def _zero_fill(buf, nrows):
    def grp(g, _):
        for r in range(L):
            buf[g * L + r, :] = jnp.zeros((L,), jnp.float32)
        return 0
    lax.fori_loop(0, nrows // L, grp, 0)


def _deg_body(edges_ref, ab_ref, deg_sh, idx_v, idx_w, ones_v, vbuf, obuf,
              si0, si1, sd0, sd1):
    c = lax.axis_index("c")
    s = lax.axis_index("s")

    def init_ones(i, _):
        ones_v[pl.ds(i * L, L)] = jnp.ones((L,), jnp.float32)
        return 0
    lax.fori_loop(0, EC // L, init_ones, 0)

    def init_zero(i, _):
        obuf[pl.ds(i * L, L)] = jnp.zeros((L,), jnp.float32)
        return 0
    lax.fori_loop(0, RC // L, init_zero, 0)

    def zero_chunk(j, _):
        cid = s + NS * j
        pltpu.sync_copy(obuf, deg_sh.at[pl.ds(cid * RC, RC)])
        return 0
    lax.fori_loop(0, _n_chunks(s), zero_chunk, 0)

    plsc.subcore_barrier()

    ebase = c * N_EDGES + s * EPT
    pltpu.async_copy(edges_ref.at[pl.ds(ebase, EC)], idx_v, si0)
    pltpu.async_copy(edges_ref.at[pl.ds(ebase + EC, EC)], idx_w, si1)

    def _deg_scat(idx):
        h = EC // 2
        d1 = pltpu.async_copy(ones_v.at[pl.ds(0, h)],
                              deg_sh.at[idx.at[pl.ds(0, h)]], sd0,
                              add=True)
        d2 = pltpu.async_copy(ones_v.at[pl.ds(h, h)],
                              deg_sh.at[idx.at[pl.ds(h, h)]], sd1,
                              add=True)
        d1.wait()
        d2.wait()

    def deg_pair(j, _):
        b0 = ebase + 2 * j * EC
        pltpu.make_async_copy(edges_ref.at[pl.ds(b0, EC)], idx_v,
                              si0).wait()
        _deg_scat(idx_v)
        pltpu.async_copy(edges_ref.at[pl.ds(b0 + 2 * EC, EC)], idx_v, si0)
        pltpu.make_async_copy(edges_ref.at[pl.ds(b0 + EC, EC)], idx_w,
                              si1).wait()
        _deg_scat(idx_w)
        @pl.when(j < N_ECHUNK // 2 - 1)
        def _():
            pltpu.async_copy(edges_ref.at[pl.ds(b0 + 3 * EC, EC)], idx_w,
                             si1)
        return 0
    lax.fori_loop(0, N_ECHUNK // 2, deg_pair, 0)
    pltpu.make_async_copy(
        edges_ref.at[pl.ds(ebase + (N_ECHUNK - 1) * EC, EC)], idx_v,
        si0).wait()
    _deg_scat(idx_v)

    plsc.subcore_barrier()

    def conv_chunk(j, _):
        cid = s + NS * j
        pltpu.sync_copy(deg_sh.at[pl.ds(cid * RC, RC)], vbuf)
        def conv_grp(g, _):
            x = jnp.maximum(vbuf[pl.ds(g * L, L)], jnp.float32(1.0))
            obuf[pl.ds(g * L, L)] = _vrsqrt(x)
            return 0
        lax.fori_loop(0, RC // L, conv_grp, 0)
        pltpu.sync_copy(obuf, ab_ref.at[pl.ds(c * N_NODES + cid * RC, RC)])
        return 0
    lax.fori_loop(0, _n_chunks(s), conv_chunk, 0)


def _main_body(x0_ref, srcoff_ref, dst_ref, ab_ref, b_ref, bidx2_ref,
               out_ref, acc_sh, xt_hbm,
               idx_s0, idx_s1, idx_d0, idx_d1, msg0, msg1, abuf, bbuf,
               sg0, sg1, sg0b, sg1b, si0, si1):
    c = lax.axis_index("c")
    s = lax.axis_index("s")
    dchunk = msg0
    btmp = msg1.at[pl.ds(0, BC), :]
    bchunk = msg1.at[pl.ds(BC, BC), :]
    bidx_v = idx_s0.at[pl.ds(0, BC)]
    bidx_a = idx_s1.at[pl.ds(0, BPT)]
    bvals = bbuf.at[pl.ds(0, BC)]

    def p1_chunk(j, _):
        cid = s + NS * j
        n0 = cid * RC
        r0 = c * N_NODES + n0
        d1 = pltpu.async_copy(ab_ref.at[pl.ds(n0, RC)], abuf, si0)
        d2 = pltpu.async_copy(x0_ref.at[pl.ds(r0, RC), :], dchunk, si1)
        d1.wait()
        d2.wait()
        _scale_inplace(dchunk, lambda g: abuf[pl.ds(g * L, L)], RC)
        pltpu.sync_copy(dchunk, xt_hbm.at[pl.ds(r0, RC), :])
        _zero_fill(dchunk, RC)
        pltpu.sync_copy(dchunk, acc_sh.at[pl.ds(n0, RC), :])
        return 0
    lax.fori_loop(0, _n_chunks(s), p1_chunk, 0)

    for m in range(BPT // BC):
        bb = s * BPT + m * BC
        pltpu.sync_copy(bidx2_ref.at[pl.ds(c * B2 + bb, BC)], bidx_v)
        pltpu.sync_copy(x0_ref.at[bidx_v], bchunk)
        pltpu.sync_copy(bchunk, out_ref.at[pl.ds(c * B2 + bb, BC), :])

    plsc.subcore_barrier()

    sbase = s * EPT
    coff = c * N_EDGES
    for layer in range(N_LAYERS):
        pltpu.sync_copy(srcoff_ref.at[pl.ds(coff + sbase, EC)], idx_s0)
        pltpu.sync_copy(dst_ref.at[pl.ds(sbase, EC)], idx_d0)
        pltpu.async_copy(xt_hbm.at[idx_s0.at[pl.ds(0, EC // 2)]],
                         msg0.at[pl.ds(0, EC // 2), :], sg0)
        pltpu.async_copy(xt_hbm.at[idx_s0.at[pl.ds(EC // 2, EC // 2)]],
                         msg0.at[pl.ds(EC // 2, EC // 2), :], sg0b)
        pltpu.async_copy(srcoff_ref.at[pl.ds(coff + sbase + EC, EC)],
                         idx_s1, si1)
        pltpu.async_copy(dst_ref.at[pl.ds(sbase + EC, EC)], idx_d1, si1)

        def edge_body(j, _):
            base0 = sbase + 2 * j * EC
            not_last = j < N_ECHUNK // 2 - 1
            pltpu.make_async_copy(
                srcoff_ref.at[pl.ds(coff + base0 + EC, EC)], idx_s1,
                si1).wait()
            pltpu.make_async_copy(
                dst_ref.at[pl.ds(base0 + EC, EC)], idx_d1, si1).wait()
            pltpu.make_async_copy(
                xt_hbm.at[idx_s0.at[pl.ds(0, EC // 2)]],
                msg0.at[pl.ds(0, EC // 2), :], sg0).wait()
            pltpu.make_async_copy(
                xt_hbm.at[idx_s0.at[pl.ds(EC // 2, EC // 2)]],
                msg0.at[pl.ds(EC // 2, EC // 2), :], sg0b).wait()
            pltpu.async_copy(xt_hbm.at[idx_s1.at[pl.ds(0, EC // 2)]],
                             msg1.at[pl.ds(0, EC // 2), :], sg1)
            pltpu.async_copy(xt_hbm.at[idx_s1.at[pl.ds(EC // 2, EC // 2)]],
                             msg1.at[pl.ds(EC // 2, EC // 2), :], sg1b)
            pltpu.sync_copy(msg0, acc_sh.at[idx_d0], add=True)
            pltpu.async_copy(
                srcoff_ref.at[pl.ds(coff + base0 + 2 * EC, EC)],
                idx_s0, si0)
            pltpu.async_copy(dst_ref.at[pl.ds(base0 + 2 * EC, EC)],
                             idx_d0, si0)
            pltpu.make_async_copy(
                xt_hbm.at[idx_s1.at[pl.ds(0, EC // 2)]],
                msg1.at[pl.ds(0, EC // 2), :], sg1).wait()
            pltpu.make_async_copy(
                xt_hbm.at[idx_s1.at[pl.ds(EC // 2, EC // 2)]],
                msg1.at[pl.ds(EC // 2, EC // 2), :], sg1b).wait()
            pltpu.make_async_copy(
                srcoff_ref.at[pl.ds(coff + base0 + 2 * EC, EC)],
                idx_s0, si0).wait()
            pltpu.make_async_copy(
                dst_ref.at[pl.ds(base0 + 2 * EC, EC)], idx_d0, si0).wait()
            pltpu.async_copy(xt_hbm.at[idx_s0.at[pl.ds(0, EC // 2)]],
                             msg0.at[pl.ds(0, EC // 2), :], sg0)
            pltpu.async_copy(xt_hbm.at[idx_s0.at[pl.ds(EC // 2, EC // 2)]],
                             msg0.at[pl.ds(EC // 2, EC // 2), :], sg0b)
            pltpu.sync_copy(msg1, acc_sh.at[idx_d1], add=True)
            @pl.when(not_last)
            def _():
                pltpu.async_copy(
                    srcoff_ref.at[pl.ds(coff + base0 + 3 * EC, EC)],
                    idx_s1, si1)
                pltpu.async_copy(dst_ref.at[pl.ds(base0 + 3 * EC, EC)],
                                 idx_d1, si1)
            return 0
        lax.fori_loop(0, N_ECHUNK // 2, edge_body, 0)
        pltpu.make_async_copy(
            xt_hbm.at[idx_s0.at[pl.ds(0, EC // 2)]],
            msg0.at[pl.ds(0, EC // 2), :], sg0).wait()
        pltpu.make_async_copy(
            xt_hbm.at[idx_s0.at[pl.ds(EC // 2, EC // 2)]],
            msg0.at[pl.ds(EC // 2, EC // 2), :], sg0b).wait()
        pltpu.sync_copy(msg0, acc_sh.at[idx_d0], add=True)

        plsc.subcore_barrier()

        pltpu.sync_copy(bidx2_ref.at[pl.ds(s * BPT, BPT)], bidx_a)
        for m in range(BPT // BC):
            bb = s * BPT + m * BC
            bidx_m = bidx_a.at[pl.ds(m * BC, BC)]
            d1 = pltpu.async_copy(acc_sh.at[bidx_m], btmp, sg0)
            d2 = pltpu.async_copy(b_ref.at[bidx_m], bvals, sg1)
            d3 = pltpu.async_copy(out_ref.at[pl.ds(c * B2 + bb, BC), :],
                                  bchunk, si0)
            d1.wait()
            d2.wait()
            d3.wait()
            def bacc_grp(g, _):
                b16 = bvals[pl.ds(g * L, L)]
                for r in range(L):
                    row = bchunk[g * L + r, :] + btmp[g * L + r, :] * b16[r]
                    if layer == N_LAYERS - 1:
                        row = row * jnp.float32(0.25)
                    bchunk[g * L + r, :] = row
                return 0
            lax.fori_loop(0, BC // L, bacc_grp, 0)
            pltpu.sync_copy(bchunk, out_ref.at[pl.ds(c * B2 + bb, BC), :])

        plsc.subcore_barrier()

        if layer < N_LAYERS - 1:
            def dense_chunk(j, _):
                cid = s + NS * j
                n0 = cid * RC
                r0 = c * N_NODES + n0
                d1 = pltpu.async_copy(ab_ref.at[pl.ds(n0, RC)], abuf,
                                      si0)
                d2 = pltpu.async_copy(b_ref.at[pl.ds(n0, RC)], bbuf, si1)
                d3 = pltpu.async_copy(acc_sh.at[pl.ds(n0, RC), :], dchunk,
                                      sg0)
                d1.wait()
                d2.wait()
                d3.wait()
                _scale_inplace(
                    dchunk,
                    lambda g: abuf[pl.ds(g * L, L)] * bbuf[pl.ds(g * L, L)],
                    RC)
                pltpu.sync_copy(dchunk, xt_hbm.at[pl.ds(r0, RC), :])
                _zero_fill(dchunk, RC)
                pltpu.sync_copy(dchunk, acc_sh.at[pl.ds(n0, RC), :])
                return 0
            lax.fori_loop(0, _n_chunks(s), dense_chunk, 0)

            plsc.subcore_barrier()


@jax.jit
def _run(x0flat, edges_flat, srcoff, dst, bidx2):
    mesh = plsc.VectorSubcoreMesh(core_axis_name="c", subcore_axis_name="s")
    params = pltpu.CompilerParams(use_tc_tiling_on_sc=False)

    deg_k = pl.kernel(
        _deg_body,
        out_type=jax.ShapeDtypeStruct((2 * N_NODES,), jnp.float32),
        mesh=mesh,
        compiler_params=params,
        scratch_types=[
            pltpu.VMEM_SHARED((N_NODES,), jnp.float32),
            pltpu.VMEM((EC,), jnp.int32),
            pltpu.VMEM((EC,), jnp.int32),
            pltpu.VMEM((EC,), jnp.float32),
            pltpu.VMEM((RC,), jnp.float32),
            pltpu.VMEM((RC,), jnp.float32),
            pltpu.SemaphoreType.DMA,
            pltpu.SemaphoreType.DMA,
            pltpu.SemaphoreType.DMA,
            pltpu.SemaphoreType.DMA,
        ],
    )
    ab = deg_k(edges_flat)
    a_tab = ab[:N_NODES]
    b_tab = ab[N_NODES:]

    main_k = pl.kernel(
        _main_body,
        out_type=jax.ShapeDtypeStruct((NC * B2, HD), jnp.float32),
        mesh=mesh,
        compiler_params=params,
        scratch_types=[
            pltpu.VMEM_SHARED((N_NODES, HD), jnp.float32),
            pltpu.MemorySpace.HBM((NC * N_NODES, HD), jnp.float32),
            pltpu.VMEM((EC,), jnp.int32),
            pltpu.VMEM((EC,), jnp.int32),
            pltpu.VMEM((EC,), jnp.int32),
            pltpu.VMEM((EC,), jnp.int32),
            pltpu.VMEM((EC, HD), jnp.float32),
            pltpu.VMEM((EC, HD), jnp.float32),
            pltpu.VMEM((RC,), jnp.float32),
            pltpu.VMEM((RC,), jnp.float32),
            pltpu.SemaphoreType.DMA,
            pltpu.SemaphoreType.DMA,
            pltpu.SemaphoreType.DMA,
            pltpu.SemaphoreType.DMA,
            pltpu.SemaphoreType.DMA,
            pltpu.SemaphoreType.DMA,
        ],
    )
    return main_k(x0flat, srcoff, dst, a_tab, b_tab, bidx2)


def kernel(users, items, edge_index, user_emb, item_emb):
    src = edge_index[0].astype(jnp.int32)
    dst = edge_index[1].astype(jnp.int32)
    x0 = jnp.concatenate([user_emb, item_emb], axis=0)
    x0flat = jnp.concatenate([x0[:, :HD], x0[:, HD:]], axis=0)
    edges_flat = jnp.concatenate([src, dst])
    srcoff = jnp.concatenate([src, src + N_NODES])
    bidx = jnp.concatenate([users.astype(jnp.int32),
                            items.astype(jnp.int32) + N_USERS])
    bidx2 = jnp.concatenate([bidx, bidx + N_NODES])
    out = _run(x0flat, edges_flat, srcoff, dst, bidx2)
    users_emb = jnp.concatenate([out[:BATCH], out[B2:B2 + BATCH]], axis=1)
    items_emb = jnp.concatenate([out[BATCH:B2], out[B2 + BATCH:]], axis=1)
    return (users_emb, items_emb)
